# baseline (speedup 1.0000x reference)
import functools

import jax
import jax.numpy as jnp
from jax import lax
from jax.experimental import pallas as pl
from jax.experimental.pallas import tpu as pltpu
from jax.experimental.pallas import tpu_sc as plsc

NC = 2
NS = 16
NW = NC * NS

EMBED_DIM = 32
BATCH = 16384
FIELDS = 26
KROWS = BATCH // NW
SUB = 128
NSUB = KROWS // SUB
NGRP = KROWS // 16


def _gather_body(idx_hbm, table_hbm, out_hbm,
                 idxv0, idxv1, rst0, rst1, tb0, tb1,
                 gsem0, gsem1, osem0, osem1):
    wid = lax.axis_index("s") * NC + lax.axis_index("c")
    b0 = wid * KROWS
    idxv = (idxv0, idxv1)
    rst = (rst0, rst1)
    tbuf = (tb0, tb1)
    gsem = (gsem0, gsem1)
    osem = (osem0, osem1)

    lane = lax.broadcasted_iota(jnp.int32, (16,), 0)

    def load_idx(f, p):
        pltpu.sync_copy(idx_hbm.at[f, pl.ds(b0, KROWS)], idxv[p])

    def fire_gather(p):
        for j in range(NSUB):
            pltpu.async_copy(
                table_hbm.at[idxv[p].at[pl.ds(j * SUB, SUB)]],
                rst[p].at[pl.ds(j * SUB, SUB)],
                gsem[p])

    def drain_gather(p):
        pltpu.make_async_copy(
            table_hbm.at[idxv[p]], rst[p], gsem[p]).wait()

    def transpose(p):
        def g_body(g, carry):
            row = g * 16 + lane
            for e in range(EMBED_DIM):
                col = jnp.full((16,), e, jnp.int32)
                vals = plsc.load_gather(rst[p], [row, col])
                tbuf[p][e, pl.ds(g * 16, 16)] = vals
            return carry
        lax.fori_loop(0, NGRP, g_body, 0)

    def fire_out(f, p):
        pltpu.async_copy(
            tbuf[p], out_hbm.at[f, :, pl.ds(b0, KROWS)], osem[p])

    def drain_out(p):
        pltpu.make_async_copy(
            tbuf[p], out_hbm.at[0, :, pl.ds(b0, KROWS)], osem[p]).wait()

    load_idx(0, 0)
    fire_gather(0)
    load_idx(1, 1)
    fire_gather(1)

    def pair_body(f2, carry):
        f = f2 * 2
        for p in (0, 1):
            fp = f + p
            drain_gather(p)
            pl.when(f2 >= 1)(lambda: drain_out(p))
            transpose(p)
            fire_out(fp, p)

            def prefetch(fp=fp, p=p):
                load_idx(fp + 2, p)
                fire_gather(p)
            pl.when(fp + 2 < FIELDS)(prefetch)
        return carry

    lax.fori_loop(0, FIELDS // 2, pair_body, 0)
    drain_out(0)
    drain_out(1)


@jax.jit
def _sc_gather(idx_t, table):
    mesh = plsc.VectorSubcoreMesh(
        core_axis_name="c", subcore_axis_name="s",
        num_cores=NC, num_subcores=NS)
    return pl.kernel(
        _gather_body,
        out_type=jax.ShapeDtypeStruct((FIELDS, EMBED_DIM, BATCH), jnp.float32),
        mesh=mesh,
        scratch_types=[
            pltpu.VMEM((KROWS,), jnp.int32),
            pltpu.VMEM((KROWS,), jnp.int32),
            pltpu.VMEM((KROWS, EMBED_DIM), jnp.float32),
            pltpu.VMEM((KROWS, EMBED_DIM), jnp.float32),
            pltpu.VMEM((EMBED_DIM, KROWS), jnp.float32),
            pltpu.VMEM((EMBED_DIM, KROWS), jnp.float32),
            pltpu.SemaphoreType.DMA,
            pltpu.SemaphoreType.DMA,
            pltpu.SemaphoreType.DMA,
            pltpu.SemaphoreType.DMA,
        ],
        compiler_params=pltpu.CompilerParams(
            use_tc_tiling_on_sc=False, needs_layout_passes=False),
    )(idx_t, table)


def kernel(idx, table):
    idx_t = idx.T.astype(jnp.int32)
    out_t = _sc_gather(idx_t, table)
    return out_t.transpose(2, 0, 1)


# speedup vs baseline: 1.0417x; 1.0417x over previous
import jax
import jax.numpy as jnp
from jax import lax
from jax.experimental import pallas as pl
from jax.experimental.pallas import tpu as pltpu
from jax.experimental.pallas import tpu_sc as plsc

NC = 2
NS = 16
NW = NC * NS

EMBED_DIM = 32
BATCH = 16384
FIELDS = 26
KROWS = 256
NBLK = BATCH // KROWS
BLK_PER_W = NBLK // NW
SUB = 128
NSUB = KROWS // SUB
NGRP = KROWS // 16


def _gather_body(idx_hbm, table_hbm, out_hbm,
                 idxv0, idxv1, srv0, srv1, sst0, sst1, tb0, tb1,
                 gsem0, gsem1, osem0, osem1):
    wid = lax.axis_index("s") * NC + lax.axis_index("c")
    idxv = (idxv0, idxv1)
    srv = (srv0, srv1)
    sst = (sst0, sst1)
    tbuf = (tb0, tb1)
    gsem = (gsem0, gsem1)
    osem = (osem0, osem1)
    b0s = (wid * KROWS, (wid + NW) * KROWS)

    lane = lax.broadcasted_iota(jnp.int32, (16,), 0)

    def load_idx(f, p):
        pltpu.sync_copy(idx_hbm.at[f, pl.ds(b0s[p], KROWS)], idxv[p])

    def compute_srows(p):
        def s_body(g, carry):
            v = idxv[p][pl.ds(g * 16, 16)]
            srv[p][pl.ds(g * 16, 16)] = lax.shift_right_logical(v, 2)
            return carry
        lax.fori_loop(0, NGRP, s_body, 0)

    def fire_gather(p):
        for j in range(NSUB):
            pltpu.async_copy(
                table_hbm.at[srv[p].at[pl.ds(j * SUB, SUB)]],
                sst[p].at[pl.ds(j * SUB, SUB)],
                gsem[p])

    def drain_gather(p):
        pltpu.make_async_copy(
            table_hbm.at[srv[p]], sst[p], gsem[p]).wait()

    def extract(p):
        def g_body(g, carry):
            iv = idxv[p][pl.ds(g * 16, 16)]
            row = g * 16 + lane
            qcol = lax.shift_left(lax.bitwise_and(iv, 3), 5)
            for e in range(EMBED_DIM):
                vals = plsc.load_gather(sst[p], [row, qcol + e])
                tbuf[p][e, pl.ds(g * 16, 16)] = vals
            return carry
        lax.fori_loop(0, NGRP, g_body, 0)

    def fire_out(f, p):
        pltpu.async_copy(
            tbuf[p], out_hbm.at[f, :, pl.ds(b0s[p], KROWS)], osem[p])

    def drain_out(p):
        pltpu.make_async_copy(
            tbuf[p], out_hbm.at[0, :, pl.ds(b0s[p], KROWS)], osem[p]).wait()

    for p in (0, 1):
        load_idx(0, p)
        compute_srows(p)
        fire_gather(p)

    def field_body(f, carry):
        for p in (0, 1):
            drain_gather(p)
            pl.when(f >= 1)(lambda: drain_out(p))
            extract(p)
            fire_out(f, p)

            def prefetch(p=p):
                load_idx(f + 1, p)
                compute_srows(p)
                fire_gather(p)
            pl.when(f + 1 < FIELDS)(prefetch)
        return carry

    lax.fori_loop(0, FIELDS, field_body, 0)
    drain_out(0)
    drain_out(1)


@jax.jit
def _sc_gather(idx_t, table_sr):
    mesh = plsc.VectorSubcoreMesh(
        core_axis_name="c", subcore_axis_name="s",
        num_cores=NC, num_subcores=NS)
    return pl.kernel(
        _gather_body,
        out_type=jax.ShapeDtypeStruct((FIELDS, EMBED_DIM, BATCH), jnp.float32),
        mesh=mesh,
        scratch_types=[
            pltpu.VMEM((KROWS,), jnp.int32),
            pltpu.VMEM((KROWS,), jnp.int32),
            pltpu.VMEM((KROWS,), jnp.int32),
            pltpu.VMEM((KROWS,), jnp.int32),
            pltpu.VMEM((KROWS, 128), jnp.float32),
            pltpu.VMEM((KROWS, 128), jnp.float32),
            pltpu.VMEM((EMBED_DIM, KROWS), jnp.float32),
            pltpu.VMEM((EMBED_DIM, KROWS), jnp.float32),
            pltpu.SemaphoreType.DMA,
            pltpu.SemaphoreType.DMA,
            pltpu.SemaphoreType.DMA,
            pltpu.SemaphoreType.DMA,
        ],
        compiler_params=pltpu.CompilerParams(
            use_tc_tiling_on_sc=True, needs_layout_passes=False),
    )(idx_t, table_sr)


def kernel(idx, table):
    idx_t = idx.T.astype(jnp.int32)
    table_sr = table.reshape(250000, 128)
    out_t = _sc_gather(idx_t, table_sr)
    return out_t.transpose(2, 0, 1)


# speedup vs baseline: 1.2335x; 1.1841x over previous
import jax
import jax.numpy as jnp
from jax import lax
from jax.experimental import pallas as pl
from jax.experimental.pallas import tpu as pltpu
from jax.experimental.pallas import tpu_sc as plsc

NC = 2
NS = 16
NW = NC * NS

EMBED_DIM = 32
BATCH = 16384
FIELDS = 26
KROWS = 256
NBLK = BATCH // KROWS
BLK_PER_W = NBLK // NW
SUB = 128
NSUB = KROWS // SUB
NGRP = KROWS // 16


def _gather_body(idx_hbm, table_hbm, out_hbm,
                 idxv0, idxv1, srv0, srv1, sst0, sst1, tb0, tb1,
                 gsem0, gsem1, osem0, osem1):
    wid = lax.axis_index("s") * NC + lax.axis_index("c")
    idxv = (idxv0, idxv1)
    srv = (srv0, srv1)
    sst = (sst0, sst1)
    tbuf = (tb0, tb1)
    gsem = (gsem0, gsem1)
    osem = (osem0, osem1)
    b0s = (wid * KROWS, (wid + NW) * KROWS)

    lane = lax.broadcasted_iota(jnp.int32, (16,), 0)

    def load_idx(f, p):
        pltpu.sync_copy(idx_hbm.at[f, pl.ds(b0s[p], KROWS)], idxv[p])

    def compute_srows(p):
        @plsc.parallel_loop(0, NGRP, unroll=4)
        def _(g):
            v = idxv[p][pl.ds(g * 16, 16)]
            srv[p][pl.ds(g * 16, 16)] = lax.shift_right_logical(v, 2)

    def fire_gather(p):
        for j in range(NSUB):
            pltpu.async_copy(
                table_hbm.at[srv[p].at[pl.ds(j * SUB, SUB)]],
                sst[p].at[pl.ds(j * SUB, SUB)],
                gsem[p])

    def drain_gather(p):
        pltpu.make_async_copy(
            table_hbm.at[srv[p]], sst[p], gsem[p]).wait()

    def extract(p):
        @plsc.parallel_loop(0, NGRP, unroll=2)
        def _(g):
            iv = idxv[p][pl.ds(g * 16, 16)]
            row = g * 16 + lane
            qcol = lax.shift_left(lax.bitwise_and(iv, 3), 5)
            for e in range(EMBED_DIM):
                vals = plsc.load_gather(sst[p], [row, qcol + e])
                tbuf[p][e, pl.ds(g * 16, 16)] = vals

    def fire_out(f, p):
        pltpu.async_copy(
            tbuf[p], out_hbm.at[f, :, pl.ds(b0s[p], KROWS)], osem[p])

    def drain_out(p):
        pltpu.make_async_copy(
            tbuf[p], out_hbm.at[0, :, pl.ds(b0s[p], KROWS)], osem[p]).wait()

    for p in (0, 1):
        load_idx(0, p)
        compute_srows(p)
        fire_gather(p)

    def field_body(f, carry):
        for p in (0, 1):
            drain_gather(p)
            pl.when(f >= 1)(lambda: drain_out(p))
            extract(p)
            fire_out(f, p)

            def prefetch(p=p):
                load_idx(f + 1, p)
                compute_srows(p)
                fire_gather(p)
            pl.when(f + 1 < FIELDS)(prefetch)
        return carry

    lax.fori_loop(0, FIELDS, field_body, 0)
    drain_out(0)
    drain_out(1)


@jax.jit
def _sc_gather(idx_t, table_sr):
    mesh = plsc.VectorSubcoreMesh(
        core_axis_name="c", subcore_axis_name="s",
        num_cores=NC, num_subcores=NS)
    return pl.kernel(
        _gather_body,
        out_type=jax.ShapeDtypeStruct((FIELDS, EMBED_DIM, BATCH), jnp.float32),
        mesh=mesh,
        scratch_types=[
            pltpu.VMEM((KROWS,), jnp.int32),
            pltpu.VMEM((KROWS,), jnp.int32),
            pltpu.VMEM((KROWS,), jnp.int32),
            pltpu.VMEM((KROWS,), jnp.int32),
            pltpu.VMEM((KROWS, 128), jnp.float32),
            pltpu.VMEM((KROWS, 128), jnp.float32),
            pltpu.VMEM((EMBED_DIM, KROWS), jnp.float32),
            pltpu.VMEM((EMBED_DIM, KROWS), jnp.float32),
            pltpu.SemaphoreType.DMA,
            pltpu.SemaphoreType.DMA,
            pltpu.SemaphoreType.DMA,
            pltpu.SemaphoreType.DMA,
        ],
        compiler_params=pltpu.CompilerParams(
            use_tc_tiling_on_sc=True, needs_layout_passes=False),
    )(idx_t, table_sr)


def kernel(idx, table):
    idx_t = idx.T.astype(jnp.int32)
    table_sr = table.reshape(250000, 128)
    out_t = _sc_gather(idx_t, table_sr)
    return out_t.transpose(2, 0, 1)
